# baseline (speedup 1.0000x reference)
import functools

import jax
import jax.numpy as jnp
from jax import lax
from jax.experimental import pallas as pl
from jax.experimental.pallas import tpu as pltpu
from jax.experimental.pallas import tpu_sc as plsc

NC = 2
NS = 16
NW = NC * NS

H = 128
CH = 128


def _mesh():
    return plsc.VectorSubcoreMesh(core_axis_name="c", subcore_axis_name="s")


def _make_row_gather(n_idx, chunk=CH):
    per_w = n_idx // NW
    nch = per_w // chunk
    assert per_w * NW == n_idx and nch * chunk == per_w and nch % 2 == 0

    @functools.partial(
        pl.kernel,
        out_type=jax.ShapeDtypeStruct((n_idx, H), jnp.float32),
        mesh=_mesh(),
        scratch_types=[
            pltpu.VMEM((chunk,), jnp.int32),
            pltpu.VMEM((chunk,), jnp.int32),
            pltpu.VMEM((chunk, H), jnp.float32),
            pltpu.VMEM((chunk, H), jnp.float32),
            pltpu.SemaphoreType.DMA,
            pltpu.SemaphoreType.DMA,
        ],
    )
    def k(table_hbm, idx_hbm, out_hbm, idx_a, idx_b, rows_a, rows_b,
          sem_a, sem_b):
        wid = lax.axis_index("s") * NC + lax.axis_index("c")
        base = wid * per_w

        pltpu.sync_copy(idx_hbm.at[pl.ds(base, chunk)], idx_a)
        pltpu.async_copy(table_hbm.at[idx_a], rows_a, sem_a)
        pltpu.sync_copy(idx_hbm.at[pl.ds(base + chunk, chunk)], idx_b)
        pltpu.async_copy(table_hbm.at[idx_b], rows_b, sem_b)

        def body(t, carry):
            c0 = 2 * t
            pltpu.make_async_copy(table_hbm.at[pl.ds(0, chunk)], rows_a,
                                  sem_a).wait()
            pltpu.sync_copy(rows_a, out_hbm.at[pl.ds(base + c0 * chunk, chunk)])
            pltpu.sync_copy(idx_hbm.at[pl.ds(base + (c0 + 2) * chunk, chunk)],
                            idx_a)
            pltpu.async_copy(table_hbm.at[idx_a], rows_a, sem_a)
            pltpu.make_async_copy(table_hbm.at[pl.ds(0, chunk)], rows_b,
                                  sem_b).wait()
            pltpu.sync_copy(rows_b,
                            out_hbm.at[pl.ds(base + (c0 + 1) * chunk, chunk)])
            pltpu.sync_copy(idx_hbm.at[pl.ds(base + (c0 + 3) * chunk, chunk)],
                            idx_b)
            pltpu.async_copy(table_hbm.at[idx_b], rows_b, sem_b)
            return carry

        if nch > 2:
            lax.fori_loop(0, nch // 2 - 1, body, 0)
        pltpu.make_async_copy(table_hbm.at[pl.ds(0, chunk)], rows_a,
                              sem_a).wait()
        pltpu.sync_copy(rows_a,
                        out_hbm.at[pl.ds(base + (nch - 2) * chunk, chunk)])
        pltpu.make_async_copy(table_hbm.at[pl.ds(0, chunk)], rows_b,
                              sem_b).wait()
        pltpu.sync_copy(rows_b,
                        out_hbm.at[pl.ds(base + (nch - 1) * chunk, chunk)])

    return k


def _make_seg_sum(n_acc, e_pad, chunk=CH):
    per_w = e_pad // NW
    nch = per_w // chunk
    rpt = n_acc // NS
    slabs = rpt // chunk
    assert per_w * NW == e_pad and nch * chunk == per_w and nch % 2 == 0
    assert rpt * NS == n_acc and slabs * chunk == rpt

    @functools.partial(
        pl.kernel,
        out_type=jax.ShapeDtypeStruct((NC, n_acc, H), jnp.float32),
        mesh=_mesh(),
        scratch_types=[
            pltpu.VMEM_SHARED((n_acc, H), jnp.float32),
            pltpu.VMEM((chunk,), jnp.int32),
            pltpu.VMEM((chunk,), jnp.int32),
            pltpu.VMEM((chunk,), jnp.int32),
            pltpu.VMEM((chunk,), jnp.int32),
            pltpu.VMEM((chunk, H), jnp.float32),
            pltpu.VMEM((chunk, H), jnp.float32),
            pltpu.SemaphoreType.DMA,
            pltpu.SemaphoreType.DMA,
            pltpu.SemaphoreType.DMA,
            pltpu.SemaphoreType.DMA,
        ],
    )
    def k(h_hbm, src_hbm, dst_hbm, zrow_hbm, acc_hbm,
          acc_s, sidx_a, didx_a, sidx_b, didx_b, rows_a, rows_b,
          sem_a, sem_b, sem_sa, sem_sb):
        cc = lax.axis_index("c")
        sc = lax.axis_index("s")
        wid = sc * NC + cc
        r0 = sc * rpt
        base = wid * per_w

        pltpu.sync_copy(zrow_hbm, rows_a)
        for j in range(slabs):
            pltpu.sync_copy(rows_a, acc_s.at[pl.ds(r0 + j * chunk, chunk)])
        plsc.subcore_barrier()

        def load_idx(c, si, di):
            off = base + c * chunk
            pltpu.sync_copy(src_hbm.at[pl.ds(off, chunk)], si)
            pltpu.sync_copy(dst_hbm.at[pl.ds(off, chunk)], di)

        load_idx(0, sidx_a, didx_a)
        pltpu.async_copy(h_hbm.at[sidx_a], rows_a, sem_a)
        load_idx(1, sidx_b, didx_b)
        pltpu.async_copy(h_hbm.at[sidx_b], rows_b, sem_b)

        def body(t, carry):
            c0 = 2 * t
            pltpu.make_async_copy(h_hbm.at[pl.ds(0, chunk)], rows_a,
                                  sem_a).wait()
            pltpu.async_copy(rows_a, acc_s.at[didx_a], sem_sa, add=True)
            pltpu.make_async_copy(h_hbm.at[pl.ds(0, chunk)], rows_b,
                                  sem_b).wait()
            pltpu.async_copy(rows_b, acc_s.at[didx_b], sem_sb, add=True)
            pltpu.make_async_copy(rows_a, acc_s.at[didx_a], sem_sa).wait()
            load_idx(c0 + 2, sidx_a, didx_a)
            pltpu.async_copy(h_hbm.at[sidx_a], rows_a, sem_a)
            pltpu.make_async_copy(rows_b, acc_s.at[didx_b], sem_sb).wait()
            load_idx(c0 + 3, sidx_b, didx_b)
            pltpu.async_copy(h_hbm.at[sidx_b], rows_b, sem_b)
            return carry

        lax.fori_loop(0, nch // 2 - 1, body, 0)
        pltpu.make_async_copy(h_hbm.at[pl.ds(0, chunk)], rows_a, sem_a).wait()
        pltpu.sync_copy(rows_a, acc_s.at[didx_a], add=True)
        pltpu.make_async_copy(h_hbm.at[pl.ds(0, chunk)], rows_b, sem_b).wait()
        pltpu.sync_copy(rows_b, acc_s.at[didx_b], add=True)
        plsc.subcore_barrier()

        for j in range(slabs):
            pltpu.sync_copy(acc_s.at[pl.ds(r0 + j * chunk, chunk)], rows_a)
            pltpu.sync_copy(rows_a, acc_hbm.at[cc, pl.ds(r0 + j * chunk, chunk)])

    return k


def _make_counts(n_acc, e_pad, chunk=CH):
    per_w = e_pad // NW
    nch = per_w // chunk
    rpt = n_acc // NS
    slabs = rpt // chunk
    assert per_w * NW == e_pad and nch * chunk == per_w and nch % 2 == 0
    assert rpt * NS == n_acc and slabs * chunk == rpt

    @functools.partial(
        pl.kernel,
        out_type=jax.ShapeDtypeStruct((2, NC, n_acc, H), jnp.float32),
        mesh=_mesh(),
        scratch_types=[
            pltpu.VMEM_SHARED((n_acc, H), jnp.float32),
            pltpu.VMEM((chunk,), jnp.int32),
            pltpu.VMEM((chunk,), jnp.int32),
            pltpu.VMEM((chunk, H), jnp.float32),
            pltpu.VMEM((chunk, H), jnp.float32),
            pltpu.SemaphoreType.DMA,
            pltpu.SemaphoreType.DMA,
        ],
    )
    def k(dst0_hbm, dst1_hbm, zrow_hbm, ones_hbm, cnt_hbm,
          cnt_s, didx_a, didx_b, ones_v, stage_v, sem_a, sem_b):
        cc = lax.axis_index("c")
        sc = lax.axis_index("s")
        wid = sc * NC + cc
        r0 = sc * rpt
        base = wid * per_w

        pltpu.sync_copy(ones_hbm, ones_v)

        for layer, dst_hbm in ((0, dst0_hbm), (1, dst1_hbm)):
            pltpu.sync_copy(zrow_hbm, stage_v)
            for j in range(slabs):
                pltpu.sync_copy(stage_v, cnt_s.at[pl.ds(r0 + j * chunk, chunk)])
            plsc.subcore_barrier()

            pltpu.sync_copy(dst_hbm.at[pl.ds(base, chunk)], didx_a)
            pltpu.async_copy(dst_hbm.at[pl.ds(base + chunk, chunk)], didx_b,
                             sem_b)

            def body(t, carry):
                c0 = 2 * t
                pltpu.sync_copy(ones_v, cnt_s.at[didx_a], add=True)
                pltpu.make_async_copy(dst_hbm.at[pl.ds(0, chunk)], didx_b,
                                      sem_b).wait()
                pltpu.async_copy(
                    dst_hbm.at[pl.ds(base + (c0 + 2) * chunk, chunk)],
                    didx_a, sem_a)
                pltpu.sync_copy(ones_v, cnt_s.at[didx_b], add=True)
                pltpu.make_async_copy(dst_hbm.at[pl.ds(0, chunk)], didx_a,
                                      sem_a).wait()
                pltpu.async_copy(
                    dst_hbm.at[pl.ds(base + (c0 + 3) * chunk, chunk)],
                    didx_b, sem_b)
                return carry

            lax.fori_loop(0, nch // 2 - 1, body, 0)
            pltpu.sync_copy(ones_v, cnt_s.at[didx_a], add=True)
            pltpu.make_async_copy(dst_hbm.at[pl.ds(0, chunk)], didx_b,
                                  sem_b).wait()
            pltpu.sync_copy(ones_v, cnt_s.at[didx_b], add=True)
            plsc.subcore_barrier()

            for j in range(slabs):
                pltpu.sync_copy(cnt_s.at[pl.ds(r0 + j * chunk, chunk)], stage_v)
                pltpu.sync_copy(
                    stage_v,
                    cnt_hbm.at[layer, cc, pl.ds(r0 + j * chunk, chunk)])

    return k


def _sage_layer_tc(h, acc, cnt, layer, W_self, W_neigh, b, relu, n):
    blk = 1000
    grid = n // blk

    def body(h_ref, aA_ref, aB_ref, cA_ref, cB_ref, ws_ref, wn_ref, b_ref,
             out_ref):
        c = cA_ref[0, 0, :, :1] + cB_ref[0, 0, :, :1]
        m = (aA_ref[0] + aB_ref[0]) / jnp.maximum(c, 1.0)
        y = (jnp.dot(h_ref[...], ws_ref[...],
                     preferred_element_type=jnp.float32)
             + jnp.dot(m, wn_ref[...], preferred_element_type=jnp.float32)
             + b_ref[...])
        if relu:
            y = jnp.maximum(y, 0.0)
        out_ref[...] = y

    row_spec = pl.BlockSpec((blk, H), lambda i: (i, 0))
    accA_spec = pl.BlockSpec((1, blk, H), lambda i: (0, i, 0))
    accB_spec = pl.BlockSpec((1, blk, H), lambda i: (1, i, 0))
    cntA_spec = pl.BlockSpec((1, 1, blk, H), lambda i: (layer, 0, i, 0))
    cntB_spec = pl.BlockSpec((1, 1, blk, H), lambda i: (layer, 1, i, 0))
    full = pl.BlockSpec((H, H), lambda i: (0, 0))
    bias = pl.BlockSpec((1, H), lambda i: (0, 0))
    return pl.pallas_call(
        body,
        grid=(grid,),
        in_specs=[row_spec, accA_spec, accB_spec, cntA_spec, cntB_spec,
                  full, full, bias],
        out_specs=row_spec,
        out_shape=jax.ShapeDtypeStruct((n, H), jnp.float32),
    )(h, acc, acc, cnt, cnt, W_self, W_neigh, b.reshape(1, H))


def _decoder_tc(rows, p, W1, b1, W2, b2, W3p, b3p):
    blk = 1024
    nblk = p // blk
    grid = 2 * nblk

    def body(a_ref, b_ref, w1_ref, b1_ref, w2_ref, b2_ref, w3_ref, b3_ref,
             out_ref):
        z = a_ref[...] * b_ref[...]
        z = jnp.maximum(
            jnp.dot(z, w1_ref[...], preferred_element_type=jnp.float32)
            + b1_ref[...], 0.0)
        z = jnp.maximum(
            jnp.dot(z, w2_ref[...], preferred_element_type=jnp.float32)
            + b2_ref[...], 0.0)
        out_ref[...] = (jnp.dot(z, w3_ref[...],
                                preferred_element_type=jnp.float32)
                        + b3_ref[...])

    def a_map(g):
        return (jnp.where(g < nblk, g, g + nblk), 0)

    def b_map(g):
        return (jnp.where(g < nblk, g + nblk, g + 2 * nblk), 0)

    full = pl.BlockSpec((H, H), lambda g: (0, 0))
    bias = pl.BlockSpec((1, H), lambda g: (0, 0))
    return pl.pallas_call(
        body,
        grid=(grid,),
        in_specs=[pl.BlockSpec((blk, H), a_map),
                  pl.BlockSpec((blk, H), b_map),
                  full, bias, full, bias, full, bias],
        out_specs=pl.BlockSpec((blk, H), lambda g: (g, 0)),
        out_shape=jax.ShapeDtypeStruct((2 * p, H), jnp.float32),
    )(rows, rows, W1, b1.reshape(1, H), W2, b2.reshape(1, H), W3p, b3p)


def kernel(x, edge_index0, edge_index1, pos_edges, neg_edges, emb,
           W_self0, W_neigh0, b0, W_self1, W_neigh1, b1,
           dec_W1, dec_b1, dec_W2, dec_b2, dec_W3, dec_b3):
    n = x.shape[0]
    e = edge_index0.shape[1]
    p = pos_edges.shape[1]

    x = x.astype(jnp.int32)
    ei0 = edge_index0.astype(jnp.int32)
    ei1 = edge_index1.astype(jnp.int32)

    q = 2 * CH * NW
    n_pad = ((n + q - 1) // q) * q
    x_pad = jnp.concatenate(
        [x, (jnp.arange(n_pad - n, dtype=jnp.int32) % n)])
    h0 = _make_row_gather(n_pad)(emb, x_pad)

    qh = CH * NS
    n_acc = (n // qh + 1) * qh
    e_pad = ((e + q - 1) // q) * q
    src_fill = jnp.arange(e_pad - e, dtype=jnp.int32) % n
    dst_fill = n + (jnp.arange(e_pad - e, dtype=jnp.int32) % (n_acc - n))
    src0 = jnp.concatenate([ei0[0], src_fill])
    dst0 = jnp.concatenate([ei0[1], dst_fill])
    src1 = jnp.concatenate([ei1[0], src_fill])
    dst1 = jnp.concatenate([ei1[1], dst_fill])

    zrow = jnp.zeros((CH, H), jnp.float32)
    ones = jnp.ones((CH, H), jnp.float32)

    cnt = _make_counts(n_acc, e_pad)(dst0, dst1, zrow, ones)
    seg = _make_seg_sum(n_acc, e_pad)

    acc0 = seg(h0, src0, dst0, zrow)
    h1 = _sage_layer_tc(h0, acc0, cnt, 0, W_self0, W_neigh0, b0, True, n)

    acc1 = seg(h1, src1, dst1, zrow)
    h2 = _sage_layer_tc(h1, acc1, cnt, 1, W_self1, W_neigh1, b1, False, n)

    idx_all = jnp.concatenate([pos_edges[0], pos_edges[1],
                               neg_edges[0], neg_edges[1]]).astype(jnp.int32)
    rows = _make_row_gather(4 * p)(h2, idx_all)

    W3p = jnp.pad(dec_W3, ((0, 0), (0, H - 1)))
    b3p = jnp.pad(dec_b3, (0, H - 1)).reshape(1, H)
    out = _decoder_tc(rows, p, dec_W1, dec_b1, dec_W2, dec_b2, W3p, b3p)

    h_pos = out[:p, :1]
    h_neg = out[p:2 * p, :1]
    return (h_pos, h_neg)


# speedup vs baseline: 1.0013x; 1.0013x over previous
import functools

import jax
import jax.numpy as jnp
from jax import lax
from jax.experimental import pallas as pl
from jax.experimental.pallas import tpu as pltpu
from jax.experimental.pallas import tpu_sc as plsc

NC = 2
NS = 16
NW = NC * NS

H = 128
CH = 128


def _mesh():
    return plsc.VectorSubcoreMesh(core_axis_name="c", subcore_axis_name="s")


def _make_row_gather(n_idx, chunk=CH):
    per_w = n_idx // NW
    nch = per_w // chunk
    assert per_w * NW == n_idx and nch * chunk == per_w and nch % 2 == 0

    @functools.partial(
        pl.kernel,
        out_type=jax.ShapeDtypeStruct((n_idx, H), jnp.float32),
        mesh=_mesh(),
        scratch_types=[
            pltpu.VMEM((chunk,), jnp.int32),
            pltpu.VMEM((chunk,), jnp.int32),
            pltpu.VMEM((chunk, H), jnp.float32),
            pltpu.VMEM((chunk, H), jnp.float32),
            pltpu.SemaphoreType.DMA,
            pltpu.SemaphoreType.DMA,
        ],
    )
    def k(table_hbm, idx_hbm, out_hbm, idx_a, idx_b, rows_a, rows_b,
          sem_a, sem_b):
        wid = lax.axis_index("s") * NC + lax.axis_index("c")
        base = wid * per_w

        pltpu.sync_copy(idx_hbm.at[pl.ds(base, chunk)], idx_a)
        pltpu.async_copy(table_hbm.at[idx_a], rows_a, sem_a)
        pltpu.sync_copy(idx_hbm.at[pl.ds(base + chunk, chunk)], idx_b)
        pltpu.async_copy(table_hbm.at[idx_b], rows_b, sem_b)

        def body(t, carry):
            c0 = 2 * t
            pltpu.make_async_copy(table_hbm.at[pl.ds(0, chunk)], rows_a,
                                  sem_a).wait()
            pltpu.sync_copy(rows_a, out_hbm.at[pl.ds(base + c0 * chunk, chunk)])
            pltpu.sync_copy(idx_hbm.at[pl.ds(base + (c0 + 2) * chunk, chunk)],
                            idx_a)
            pltpu.async_copy(table_hbm.at[idx_a], rows_a, sem_a)
            pltpu.make_async_copy(table_hbm.at[pl.ds(0, chunk)], rows_b,
                                  sem_b).wait()
            pltpu.sync_copy(rows_b,
                            out_hbm.at[pl.ds(base + (c0 + 1) * chunk, chunk)])
            pltpu.sync_copy(idx_hbm.at[pl.ds(base + (c0 + 3) * chunk, chunk)],
                            idx_b)
            pltpu.async_copy(table_hbm.at[idx_b], rows_b, sem_b)
            return carry

        if nch > 2:
            lax.fori_loop(0, nch // 2 - 1, body, 0)
        pltpu.make_async_copy(table_hbm.at[pl.ds(0, chunk)], rows_a,
                              sem_a).wait()
        pltpu.sync_copy(rows_a,
                        out_hbm.at[pl.ds(base + (nch - 2) * chunk, chunk)])
        pltpu.make_async_copy(table_hbm.at[pl.ds(0, chunk)], rows_b,
                              sem_b).wait()
        pltpu.sync_copy(rows_b,
                        out_hbm.at[pl.ds(base + (nch - 1) * chunk, chunk)])

    return k


def _make_seg_sum(n_acc, e_pad, chunk=CH):
    per_w = e_pad // NW
    nch = per_w // chunk
    rpt = n_acc // NS
    slabs = rpt // chunk
    assert per_w * NW == e_pad and nch * chunk == per_w and nch % 2 == 0
    assert rpt * NS == n_acc and slabs * chunk == rpt

    @functools.partial(
        pl.kernel,
        out_type=jax.ShapeDtypeStruct((NC, n_acc, H), jnp.float32),
        mesh=_mesh(),
        scratch_types=[
            pltpu.VMEM_SHARED((n_acc, H), jnp.float32),
            pltpu.VMEM((chunk,), jnp.int32),
            pltpu.VMEM((chunk,), jnp.int32),
            pltpu.VMEM((chunk,), jnp.int32),
            pltpu.VMEM((chunk,), jnp.int32),
            pltpu.VMEM((chunk, H), jnp.float32),
            pltpu.VMEM((chunk, H), jnp.float32),
            pltpu.SemaphoreType.DMA,
            pltpu.SemaphoreType.DMA,
            pltpu.SemaphoreType.DMA,
            pltpu.SemaphoreType.DMA,
        ],
    )
    def k(h_hbm, src_hbm, dst_hbm, zrow_hbm, acc_hbm,
          acc_s, sidx_a, didx_a, sidx_b, didx_b, rows_a, rows_b,
          sem_a, sem_b, sem_sa, sem_sb):
        cc = lax.axis_index("c")
        sc = lax.axis_index("s")
        wid = sc * NC + cc
        r0 = sc * rpt
        base = wid * per_w

        pltpu.sync_copy(zrow_hbm, rows_a)
        for j in range(slabs):
            pltpu.sync_copy(rows_a, acc_s.at[pl.ds(r0 + j * chunk, chunk)])
        plsc.subcore_barrier()

        def load_idx(c, si, di):
            off = base + c * chunk
            pltpu.sync_copy(src_hbm.at[pl.ds(off, chunk)], si)
            pltpu.sync_copy(dst_hbm.at[pl.ds(off, chunk)], di)

        load_idx(0, sidx_a, didx_a)
        pltpu.async_copy(h_hbm.at[sidx_a], rows_a, sem_a)
        load_idx(1, sidx_b, didx_b)
        pltpu.async_copy(h_hbm.at[sidx_b], rows_b, sem_b)

        def body(t, carry):
            c0 = 2 * t
            pltpu.make_async_copy(h_hbm.at[pl.ds(0, chunk)], rows_a,
                                  sem_a).wait()
            pltpu.sync_copy(rows_a, acc_s.at[didx_a], add=True)
            load_idx(c0 + 2, sidx_a, didx_a)
            pltpu.async_copy(h_hbm.at[sidx_a], rows_a, sem_a)
            pltpu.make_async_copy(h_hbm.at[pl.ds(0, chunk)], rows_b,
                                  sem_b).wait()
            pltpu.sync_copy(rows_b, acc_s.at[didx_b], add=True)
            load_idx(c0 + 3, sidx_b, didx_b)
            pltpu.async_copy(h_hbm.at[sidx_b], rows_b, sem_b)
            return carry

        lax.fori_loop(0, nch // 2 - 1, body, 0)
        pltpu.make_async_copy(h_hbm.at[pl.ds(0, chunk)], rows_a, sem_a).wait()
        pltpu.sync_copy(rows_a, acc_s.at[didx_a], add=True)
        pltpu.make_async_copy(h_hbm.at[pl.ds(0, chunk)], rows_b, sem_b).wait()
        pltpu.sync_copy(rows_b, acc_s.at[didx_b], add=True)
        plsc.subcore_barrier()

        for j in range(slabs):
            pltpu.sync_copy(acc_s.at[pl.ds(r0 + j * chunk, chunk)], rows_a)
            pltpu.sync_copy(rows_a, acc_hbm.at[cc, pl.ds(r0 + j * chunk, chunk)])

    return k


def _make_pre(n_pad, n_acc, e_pad, chunk=CH):
    g_per_w = n_pad // NW
    gnch = g_per_w // chunk
    per_w = e_pad // NW
    nch = per_w // chunk
    rpt = n_acc // NS
    slabs = rpt // chunk
    assert g_per_w * NW == n_pad and gnch * chunk == g_per_w and gnch % 2 == 0
    assert per_w * NW == e_pad and nch * chunk == per_w and nch % 2 == 0
    assert rpt * NS == n_acc and slabs * chunk == rpt

    @functools.partial(
        pl.kernel,
        out_type=(
            jax.ShapeDtypeStruct((n_pad, H), jnp.float32),
            jax.ShapeDtypeStruct((2, NC, n_acc, H), jnp.float32),
        ),
        mesh=_mesh(),
        scratch_types=[
            pltpu.VMEM_SHARED((n_acc, H), jnp.float32),
            pltpu.VMEM((chunk,), jnp.int32),
            pltpu.VMEM((chunk,), jnp.int32),
            pltpu.VMEM((chunk, H), jnp.float32),
            pltpu.VMEM((chunk, H), jnp.float32),
            pltpu.SemaphoreType.DMA,
            pltpu.SemaphoreType.DMA,
        ],
    )
    def k(emb_hbm, xidx_hbm, dst0_hbm, dst1_hbm, zrow_hbm, ones_hbm,
          h0_hbm, cnt_hbm,
          cnt_s, didx_a, didx_b, ones_v, stage_v, sem_a, sem_b):
        cc = lax.axis_index("c")
        sc = lax.axis_index("s")
        wid = sc * NC + cc
        r0 = sc * rpt
        base = wid * per_w

        gbase = wid * g_per_w
        rows_a, rows_b = ones_v, stage_v

        pltpu.sync_copy(xidx_hbm.at[pl.ds(gbase, chunk)], didx_a)
        pltpu.async_copy(emb_hbm.at[didx_a], rows_a, sem_a)
        pltpu.sync_copy(xidx_hbm.at[pl.ds(gbase + chunk, chunk)], didx_b)
        pltpu.async_copy(emb_hbm.at[didx_b], rows_b, sem_b)

        def gbody(t, carry):
            c0 = 2 * t
            pltpu.make_async_copy(emb_hbm.at[pl.ds(0, chunk)], rows_a,
                                  sem_a).wait()
            pltpu.sync_copy(rows_a,
                            h0_hbm.at[pl.ds(gbase + c0 * chunk, chunk)])
            pltpu.sync_copy(
                xidx_hbm.at[pl.ds(gbase + (c0 + 2) * chunk, chunk)], didx_a)
            pltpu.async_copy(emb_hbm.at[didx_a], rows_a, sem_a)
            pltpu.make_async_copy(emb_hbm.at[pl.ds(0, chunk)], rows_b,
                                  sem_b).wait()
            pltpu.sync_copy(
                rows_b, h0_hbm.at[pl.ds(gbase + (c0 + 1) * chunk, chunk)])
            pltpu.sync_copy(
                xidx_hbm.at[pl.ds(gbase + (c0 + 3) * chunk, chunk)], didx_b)
            pltpu.async_copy(emb_hbm.at[didx_b], rows_b, sem_b)
            return carry

        if gnch > 2:
            lax.fori_loop(0, gnch // 2 - 1, gbody, 0)
        pltpu.make_async_copy(emb_hbm.at[pl.ds(0, chunk)], rows_a,
                              sem_a).wait()
        pltpu.sync_copy(rows_a,
                        h0_hbm.at[pl.ds(gbase + (gnch - 2) * chunk, chunk)])
        pltpu.make_async_copy(emb_hbm.at[pl.ds(0, chunk)], rows_b,
                              sem_b).wait()
        pltpu.sync_copy(rows_b,
                        h0_hbm.at[pl.ds(gbase + (gnch - 1) * chunk, chunk)])

        pltpu.sync_copy(ones_hbm, ones_v)

        for layer, dst_hbm in ((0, dst0_hbm), (1, dst1_hbm)):
            pltpu.sync_copy(zrow_hbm, stage_v)
            for j in range(slabs):
                pltpu.sync_copy(stage_v, cnt_s.at[pl.ds(r0 + j * chunk, chunk)])
            plsc.subcore_barrier()

            pltpu.sync_copy(dst_hbm.at[pl.ds(base, chunk)], didx_a)
            pltpu.async_copy(dst_hbm.at[pl.ds(base + chunk, chunk)], didx_b,
                             sem_b)

            def body(t, carry):
                c0 = 2 * t
                pltpu.sync_copy(ones_v, cnt_s.at[didx_a], add=True)
                pltpu.make_async_copy(dst_hbm.at[pl.ds(0, chunk)], didx_b,
                                      sem_b).wait()
                pltpu.async_copy(
                    dst_hbm.at[pl.ds(base + (c0 + 2) * chunk, chunk)],
                    didx_a, sem_a)
                pltpu.sync_copy(ones_v, cnt_s.at[didx_b], add=True)
                pltpu.make_async_copy(dst_hbm.at[pl.ds(0, chunk)], didx_a,
                                      sem_a).wait()
                pltpu.async_copy(
                    dst_hbm.at[pl.ds(base + (c0 + 3) * chunk, chunk)],
                    didx_b, sem_b)
                return carry

            lax.fori_loop(0, nch // 2 - 1, body, 0)
            pltpu.sync_copy(ones_v, cnt_s.at[didx_a], add=True)
            pltpu.make_async_copy(dst_hbm.at[pl.ds(0, chunk)], didx_b,
                                  sem_b).wait()
            pltpu.sync_copy(ones_v, cnt_s.at[didx_b], add=True)
            plsc.subcore_barrier()

            for j in range(slabs):
                pltpu.sync_copy(cnt_s.at[pl.ds(r0 + j * chunk, chunk)], stage_v)
                pltpu.sync_copy(
                    stage_v,
                    cnt_hbm.at[layer, cc, pl.ds(r0 + j * chunk, chunk)])

    return k


def _sage_layer_tc(h, acc, cnt, layer, W_self, W_neigh, b, relu, n):
    blk = 1000
    grid = n // blk

    def body(h_ref, aA_ref, aB_ref, cA_ref, cB_ref, ws_ref, wn_ref, b_ref,
             out_ref):
        c = cA_ref[0, 0, :, :1] + cB_ref[0, 0, :, :1]
        m = (aA_ref[0] + aB_ref[0]) / jnp.maximum(c, 1.0)
        y = (jnp.dot(h_ref[...], ws_ref[...],
                     preferred_element_type=jnp.float32)
             + jnp.dot(m, wn_ref[...], preferred_element_type=jnp.float32)
             + b_ref[...])
        if relu:
            y = jnp.maximum(y, 0.0)
        out_ref[...] = y

    row_spec = pl.BlockSpec((blk, H), lambda i: (i, 0))
    accA_spec = pl.BlockSpec((1, blk, H), lambda i: (0, i, 0))
    accB_spec = pl.BlockSpec((1, blk, H), lambda i: (1, i, 0))
    cntA_spec = pl.BlockSpec((1, 1, blk, H), lambda i: (layer, 0, i, 0))
    cntB_spec = pl.BlockSpec((1, 1, blk, H), lambda i: (layer, 1, i, 0))
    full = pl.BlockSpec((H, H), lambda i: (0, 0))
    bias = pl.BlockSpec((1, H), lambda i: (0, 0))
    return pl.pallas_call(
        body,
        grid=(grid,),
        in_specs=[row_spec, accA_spec, accB_spec, cntA_spec, cntB_spec,
                  full, full, bias],
        out_specs=row_spec,
        out_shape=jax.ShapeDtypeStruct((n, H), jnp.float32),
    )(h, acc, acc, cnt, cnt, W_self, W_neigh, b.reshape(1, H))


def _decoder_tc(rows, p, W1, b1, W2, b2, W3p, b3p):
    blk = 1024
    nblk = p // blk
    grid = 2 * nblk

    def body(a_ref, b_ref, w1_ref, b1_ref, w2_ref, b2_ref, w3_ref, b3_ref,
             out_ref):
        z = a_ref[...] * b_ref[...]
        z = jnp.maximum(
            jnp.dot(z, w1_ref[...], preferred_element_type=jnp.float32)
            + b1_ref[...], 0.0)
        z = jnp.maximum(
            jnp.dot(z, w2_ref[...], preferred_element_type=jnp.float32)
            + b2_ref[...], 0.0)
        out_ref[...] = (jnp.dot(z, w3_ref[...],
                                preferred_element_type=jnp.float32)
                        + b3_ref[...])

    def a_map(g):
        return (jnp.where(g < nblk, g, g + nblk), 0)

    def b_map(g):
        return (jnp.where(g < nblk, g + nblk, g + 2 * nblk), 0)

    full = pl.BlockSpec((H, H), lambda g: (0, 0))
    bias = pl.BlockSpec((1, H), lambda g: (0, 0))
    return pl.pallas_call(
        body,
        grid=(grid,),
        in_specs=[pl.BlockSpec((blk, H), a_map),
                  pl.BlockSpec((blk, H), b_map),
                  full, bias, full, bias, full, bias],
        out_specs=pl.BlockSpec((blk, H), lambda g: (g, 0)),
        out_shape=jax.ShapeDtypeStruct((2 * p, H), jnp.float32),
    )(rows, rows, W1, b1.reshape(1, H), W2, b2.reshape(1, H), W3p, b3p)


def kernel(x, edge_index0, edge_index1, pos_edges, neg_edges, emb,
           W_self0, W_neigh0, b0, W_self1, W_neigh1, b1,
           dec_W1, dec_b1, dec_W2, dec_b2, dec_W3, dec_b3):
    n = x.shape[0]
    e = edge_index0.shape[1]
    p = pos_edges.shape[1]

    x = x.astype(jnp.int32)
    ei0 = edge_index0.astype(jnp.int32)
    ei1 = edge_index1.astype(jnp.int32)

    q = 2 * CH * NW
    n_pad = ((n + q - 1) // q) * q
    x_pad = jnp.concatenate(
        [x, (jnp.arange(n_pad - n, dtype=jnp.int32) % n)])

    qh = CH * NS
    n_acc = (n // qh + 1) * qh
    e_pad = ((e + q - 1) // q) * q
    src_fill = jnp.arange(e_pad - e, dtype=jnp.int32) % n
    dst_fill = n + (jnp.arange(e_pad - e, dtype=jnp.int32) % (n_acc - n))
    src0 = jnp.concatenate([ei0[0], src_fill])
    dst0 = jnp.concatenate([ei0[1], dst_fill])
    src1 = jnp.concatenate([ei1[0], src_fill])
    dst1 = jnp.concatenate([ei1[1], dst_fill])

    zrow = jnp.zeros((CH, H), jnp.float32)
    ones = jnp.ones((CH, H), jnp.float32)

    h0, cnt = _make_pre(n_pad, n_acc, e_pad)(emb, x_pad, dst0, dst1,
                                             zrow, ones)
    seg = _make_seg_sum(n_acc, e_pad)

    acc0 = seg(h0, src0, dst0, zrow)
    h1 = _sage_layer_tc(h0, acc0, cnt, 0, W_self0, W_neigh0, b0, True, n)

    acc1 = seg(h1, src1, dst1, zrow)
    h2 = _sage_layer_tc(h1, acc1, cnt, 1, W_self1, W_neigh1, b1, False, n)

    idx_all = jnp.concatenate([pos_edges[0], pos_edges[1],
                               neg_edges[0], neg_edges[1]]).astype(jnp.int32)
    rows = _make_row_gather(4 * p)(h2, idx_all)

    W3p = jnp.pad(dec_W3, ((0, 0), (0, H - 1)))
    b3p = jnp.pad(dec_b3, (0, H - 1)).reshape(1, H)
    out = _decoder_tc(rows, p, dec_W1, dec_b1, dec_W2, dec_b2, W3p, b3p)

    h_pos = out[:p, :1]
    h_neg = out[p:2 * p, :1]
    return (h_pos, h_neg)


# speedup vs baseline: 1.0071x; 1.0058x over previous
import functools

import jax
import jax.numpy as jnp
from jax import lax
from jax.experimental import pallas as pl
from jax.experimental.pallas import tpu as pltpu
from jax.experimental.pallas import tpu_sc as plsc

NC = 2
NS = 16
NW = NC * NS

H = 128
CH = 128


def _mesh():
    return plsc.VectorSubcoreMesh(core_axis_name="c", subcore_axis_name="s")


def _make_row_gather(n_idx, chunk=CH):
    per_w = n_idx // NW
    nch = per_w // chunk
    assert per_w * NW == n_idx and nch * chunk == per_w and nch % 2 == 0

    @functools.partial(
        pl.kernel,
        out_type=jax.ShapeDtypeStruct((n_idx, H), jnp.float32),
        mesh=_mesh(),
        scratch_types=[
            pltpu.VMEM((chunk,), jnp.int32),
            pltpu.VMEM((chunk,), jnp.int32),
            pltpu.VMEM((chunk, H), jnp.float32),
            pltpu.VMEM((chunk, H), jnp.float32),
            pltpu.SemaphoreType.DMA,
            pltpu.SemaphoreType.DMA,
        ],
    )
    def k(table_hbm, idx_hbm, out_hbm, idx_a, idx_b, rows_a, rows_b,
          sem_a, sem_b):
        wid = lax.axis_index("s") * NC + lax.axis_index("c")
        base = wid * per_w

        pltpu.sync_copy(idx_hbm.at[pl.ds(base, chunk)], idx_a)
        pltpu.async_copy(table_hbm.at[idx_a], rows_a, sem_a)
        pltpu.sync_copy(idx_hbm.at[pl.ds(base + chunk, chunk)], idx_b)
        pltpu.async_copy(table_hbm.at[idx_b], rows_b, sem_b)

        def body(t, carry):
            c0 = 2 * t
            pltpu.make_async_copy(table_hbm.at[pl.ds(0, chunk)], rows_a,
                                  sem_a).wait()
            pltpu.sync_copy(rows_a, out_hbm.at[pl.ds(base + c0 * chunk, chunk)])
            pltpu.sync_copy(idx_hbm.at[pl.ds(base + (c0 + 2) * chunk, chunk)],
                            idx_a)
            pltpu.async_copy(table_hbm.at[idx_a], rows_a, sem_a)
            pltpu.make_async_copy(table_hbm.at[pl.ds(0, chunk)], rows_b,
                                  sem_b).wait()
            pltpu.sync_copy(rows_b,
                            out_hbm.at[pl.ds(base + (c0 + 1) * chunk, chunk)])
            pltpu.sync_copy(idx_hbm.at[pl.ds(base + (c0 + 3) * chunk, chunk)],
                            idx_b)
            pltpu.async_copy(table_hbm.at[idx_b], rows_b, sem_b)
            return carry

        if nch > 2:
            lax.fori_loop(0, nch // 2 - 1, body, 0)
        pltpu.make_async_copy(table_hbm.at[pl.ds(0, chunk)], rows_a,
                              sem_a).wait()
        pltpu.sync_copy(rows_a,
                        out_hbm.at[pl.ds(base + (nch - 2) * chunk, chunk)])
        pltpu.make_async_copy(table_hbm.at[pl.ds(0, chunk)], rows_b,
                              sem_b).wait()
        pltpu.sync_copy(rows_b,
                        out_hbm.at[pl.ds(base + (nch - 1) * chunk, chunk)])

    return k


def _make_seg_sum(n_acc, e_pad, chunk=CH):
    per_w = e_pad // NW
    nch = per_w // chunk
    rpt = n_acc // NS
    slabs = rpt // chunk
    assert per_w * NW == e_pad and nch * chunk == per_w and nch % 2 == 0
    assert rpt * NS == n_acc and slabs * chunk == rpt

    @functools.partial(
        pl.kernel,
        out_type=jax.ShapeDtypeStruct((NC, n_acc, H), jnp.float32),
        mesh=_mesh(),
        scratch_types=[
            pltpu.VMEM_SHARED((n_acc, H), jnp.float32),
            pltpu.VMEM((chunk,), jnp.int32),
            pltpu.VMEM((chunk,), jnp.int32),
            pltpu.VMEM((chunk,), jnp.int32),
            pltpu.VMEM((chunk,), jnp.int32),
            pltpu.VMEM((chunk, H), jnp.float32),
            pltpu.VMEM((chunk, H), jnp.float32),
            pltpu.SemaphoreType.DMA,
            pltpu.SemaphoreType.DMA,
            pltpu.SemaphoreType.DMA,
            pltpu.SemaphoreType.DMA,
        ],
    )
    def k(h_hbm, src_hbm, dst_hbm, zrow_hbm, acc_hbm,
          acc_s, sidx_a, didx_a, sidx_b, didx_b, rows_a, rows_b,
          sem_a, sem_b, sem_sa, sem_sb):
        cc = lax.axis_index("c")
        sc = lax.axis_index("s")
        wid = sc * NC + cc
        r0 = sc * rpt
        base = wid * per_w

        pltpu.sync_copy(zrow_hbm, rows_a)
        for j in range(slabs):
            pltpu.sync_copy(rows_a, acc_s.at[pl.ds(r0 + j * chunk, chunk)])
        plsc.subcore_barrier()

        def load_idx(c, si, di):
            off = base + c * chunk
            pltpu.sync_copy(src_hbm.at[pl.ds(off, chunk)], si)
            pltpu.sync_copy(dst_hbm.at[pl.ds(off, chunk)], di)

        load_idx(0, sidx_a, didx_a)
        pltpu.async_copy(h_hbm.at[sidx_a], rows_a, sem_a)
        load_idx(1, sidx_b, didx_b)
        pltpu.async_copy(h_hbm.at[sidx_b], rows_b, sem_b)

        def body(t, carry):
            c0 = 2 * t
            pltpu.make_async_copy(h_hbm.at[pl.ds(0, chunk)], rows_a,
                                  sem_a).wait()
            pltpu.sync_copy(rows_a, acc_s.at[didx_a], add=True)
            load_idx(c0 + 2, sidx_a, didx_a)
            pltpu.async_copy(h_hbm.at[sidx_a], rows_a, sem_a)
            pltpu.make_async_copy(h_hbm.at[pl.ds(0, chunk)], rows_b,
                                  sem_b).wait()
            pltpu.sync_copy(rows_b, acc_s.at[didx_b], add=True)
            load_idx(c0 + 3, sidx_b, didx_b)
            pltpu.async_copy(h_hbm.at[sidx_b], rows_b, sem_b)
            return carry

        lax.fori_loop(0, nch // 2 - 1, body, 0)
        pltpu.make_async_copy(h_hbm.at[pl.ds(0, chunk)], rows_a, sem_a).wait()
        pltpu.sync_copy(rows_a, acc_s.at[didx_a], add=True)
        pltpu.make_async_copy(h_hbm.at[pl.ds(0, chunk)], rows_b, sem_b).wait()
        pltpu.sync_copy(rows_b, acc_s.at[didx_b], add=True)
        plsc.subcore_barrier()

        for j in range(slabs):
            pltpu.sync_copy(acc_s.at[pl.ds(r0 + j * chunk, chunk)], rows_a)
            pltpu.sync_copy(rows_a, acc_hbm.at[cc, pl.ds(r0 + j * chunk, chunk)])

    return k


def _make_counts(n_acc, e_pad, chunk=CH):
    per_w = e_pad // NW
    nch = per_w // chunk
    rpt = n_acc // NS
    slabs = rpt // chunk
    assert per_w * NW == e_pad and nch * chunk == per_w and nch % 2 == 0
    assert rpt * NS == n_acc and slabs * chunk == rpt

    @functools.partial(
        pl.kernel,
        out_type=jax.ShapeDtypeStruct((2, NC, n_acc, H), jnp.float32),
        mesh=_mesh(),
        scratch_types=[
            pltpu.VMEM_SHARED((n_acc, H), jnp.float32),
            pltpu.VMEM((chunk,), jnp.int32),
            pltpu.VMEM((chunk,), jnp.int32),
            pltpu.VMEM((chunk, H), jnp.float32),
            pltpu.VMEM((chunk, H), jnp.float32),
            pltpu.SemaphoreType.DMA,
            pltpu.SemaphoreType.DMA,
        ],
    )
    def k(dst0_hbm, dst1_hbm, zrow_hbm, ones_hbm, cnt_hbm,
          cnt_s, didx_a, didx_b, ones_v, stage_v, sem_a, sem_b):
        cc = lax.axis_index("c")
        sc = lax.axis_index("s")
        wid = sc * NC + cc
        r0 = sc * rpt
        base = wid * per_w

        pltpu.sync_copy(ones_hbm, ones_v)

        for layer, dst_hbm in ((0, dst0_hbm), (1, dst1_hbm)):
            pltpu.sync_copy(zrow_hbm, stage_v)
            for j in range(slabs):
                pltpu.sync_copy(stage_v, cnt_s.at[pl.ds(r0 + j * chunk, chunk)])
            plsc.subcore_barrier()

            pltpu.sync_copy(dst_hbm.at[pl.ds(base, chunk)], didx_a)
            pltpu.async_copy(dst_hbm.at[pl.ds(base + chunk, chunk)], didx_b,
                             sem_b)

            def body(t, carry):
                c0 = 2 * t
                pltpu.sync_copy(ones_v, cnt_s.at[didx_a], add=True)
                pltpu.make_async_copy(dst_hbm.at[pl.ds(0, chunk)], didx_b,
                                      sem_b).wait()
                pltpu.async_copy(
                    dst_hbm.at[pl.ds(base + (c0 + 2) * chunk, chunk)],
                    didx_a, sem_a)
                pltpu.sync_copy(ones_v, cnt_s.at[didx_b], add=True)
                pltpu.make_async_copy(dst_hbm.at[pl.ds(0, chunk)], didx_a,
                                      sem_a).wait()
                pltpu.async_copy(
                    dst_hbm.at[pl.ds(base + (c0 + 3) * chunk, chunk)],
                    didx_b, sem_b)
                return carry

            lax.fori_loop(0, nch // 2 - 1, body, 0)
            pltpu.sync_copy(ones_v, cnt_s.at[didx_a], add=True)
            pltpu.make_async_copy(dst_hbm.at[pl.ds(0, chunk)], didx_b,
                                  sem_b).wait()
            pltpu.sync_copy(ones_v, cnt_s.at[didx_b], add=True)
            plsc.subcore_barrier()

            for j in range(slabs):
                pltpu.sync_copy(cnt_s.at[pl.ds(r0 + j * chunk, chunk)], stage_v)
                pltpu.sync_copy(
                    stage_v,
                    cnt_hbm.at[layer, cc, pl.ds(r0 + j * chunk, chunk)])

    return k


def _sage_layer_tc(h, acc, cnt, layer, W_self, W_neigh, b, relu, n):
    blk = 1000
    grid = n // blk

    def body(h_ref, aA_ref, aB_ref, cA_ref, cB_ref, ws_ref, wn_ref, b_ref,
             out_ref):
        c = cA_ref[0, 0, :, :1] + cB_ref[0, 0, :, :1]
        m = (aA_ref[0] + aB_ref[0]) / jnp.maximum(c, 1.0)
        y = (jnp.dot(h_ref[...], ws_ref[...],
                     preferred_element_type=jnp.float32)
             + jnp.dot(m, wn_ref[...], preferred_element_type=jnp.float32)
             + b_ref[...])
        if relu:
            y = jnp.maximum(y, 0.0)
        out_ref[...] = y

    row_spec = pl.BlockSpec((blk, H), lambda i: (i, 0))
    accA_spec = pl.BlockSpec((1, blk, H), lambda i: (0, i, 0))
    accB_spec = pl.BlockSpec((1, blk, H), lambda i: (1, i, 0))
    cntA_spec = pl.BlockSpec((1, 1, blk, H), lambda i: (layer, 0, i, 0))
    cntB_spec = pl.BlockSpec((1, 1, blk, H), lambda i: (layer, 1, i, 0))
    full = pl.BlockSpec((H, H), lambda i: (0, 0))
    bias = pl.BlockSpec((1, H), lambda i: (0, 0))
    return pl.pallas_call(
        body,
        grid=(grid,),
        in_specs=[row_spec, accA_spec, accB_spec, cntA_spec, cntB_spec,
                  full, full, bias],
        out_specs=row_spec,
        out_shape=jax.ShapeDtypeStruct((n, H), jnp.float32),
    )(h, acc, acc, cnt, cnt, W_self, W_neigh, b.reshape(1, H))


def _decoder_tc(rows, p, W1, b1, W2, b2, W3p, b3p):
    blk = 1024
    nblk = p // blk
    grid = 2 * nblk

    def body(a_ref, b_ref, w1_ref, b1_ref, w2_ref, b2_ref, w3_ref, b3_ref,
             out_ref):
        z = a_ref[...] * b_ref[...]
        z = jnp.maximum(
            jnp.dot(z, w1_ref[...], preferred_element_type=jnp.float32)
            + b1_ref[...], 0.0)
        z = jnp.maximum(
            jnp.dot(z, w2_ref[...], preferred_element_type=jnp.float32)
            + b2_ref[...], 0.0)
        out_ref[...] = (jnp.dot(z, w3_ref[...],
                                preferred_element_type=jnp.float32)
                        + b3_ref[...])

    def a_map(g):
        return (jnp.where(g < nblk, g, g + nblk), 0)

    def b_map(g):
        return (jnp.where(g < nblk, g + nblk, g + 2 * nblk), 0)

    full = pl.BlockSpec((H, H), lambda g: (0, 0))
    bias = pl.BlockSpec((1, H), lambda g: (0, 0))
    return pl.pallas_call(
        body,
        grid=(grid,),
        in_specs=[pl.BlockSpec((blk, H), a_map),
                  pl.BlockSpec((blk, H), b_map),
                  full, bias, full, bias, full, bias],
        out_specs=pl.BlockSpec((blk, H), lambda g: (g, 0)),
        out_shape=jax.ShapeDtypeStruct((2 * p, H), jnp.float32),
    )(rows, rows, W1, b1.reshape(1, H), W2, b2.reshape(1, H), W3p, b3p)


def kernel(x, edge_index0, edge_index1, pos_edges, neg_edges, emb,
           W_self0, W_neigh0, b0, W_self1, W_neigh1, b1,
           dec_W1, dec_b1, dec_W2, dec_b2, dec_W3, dec_b3):
    n = x.shape[0]
    e = edge_index0.shape[1]
    p = pos_edges.shape[1]

    x = x.astype(jnp.int32)
    ei0 = edge_index0.astype(jnp.int32)
    ei1 = edge_index1.astype(jnp.int32)

    q = 2 * CH * NW
    n_pad = ((n + q - 1) // q) * q
    x_pad = jnp.concatenate(
        [x, (jnp.arange(n_pad - n, dtype=jnp.int32) % n)])

    qh = CH * NS
    n_acc = (n // qh + 1) * qh
    e_pad = ((e + q - 1) // q) * q
    src_fill = jnp.arange(e_pad - e, dtype=jnp.int32) % n
    dst_fill = n + (jnp.arange(e_pad - e, dtype=jnp.int32) % (n_acc - n))
    src0 = jnp.concatenate([ei0[0], src_fill])
    dst0 = jnp.concatenate([ei0[1], dst_fill])
    src1 = jnp.concatenate([ei1[0], src_fill])
    dst1 = jnp.concatenate([ei1[1], dst_fill])

    zrow = jnp.zeros((CH, H), jnp.float32)
    ones = jnp.ones((CH, H), jnp.float32)

    h0 = _make_row_gather(n_pad)(emb, x_pad)
    cnt = _make_counts(n_acc, e_pad)(dst0, dst1, zrow, ones)
    seg = _make_seg_sum(n_acc, e_pad)

    acc0 = seg(h0, src0, dst0, zrow)
    h1 = _sage_layer_tc(h0, acc0, cnt, 0, W_self0, W_neigh0, b0, True, n)

    acc1 = seg(h1, src1, dst1, zrow)
    h2 = _sage_layer_tc(h1, acc1, cnt, 1, W_self1, W_neigh1, b1, False, n)

    idx_all = jnp.concatenate([pos_edges[0], pos_edges[1],
                               neg_edges[0], neg_edges[1]]).astype(jnp.int32)
    rows = _make_row_gather(4 * p)(h2, idx_all)

    W3p = jnp.pad(dec_W3, ((0, 0), (0, H - 1)))
    b3p = jnp.pad(dec_b3, (0, H - 1)).reshape(1, H)
    out = _decoder_tc(rows, p, dec_W1, dec_b1, dec_W2, dec_b2, W3p, b3p)

    h_pos = out[:p, :1]
    h_neg = out[p:2 * p, :1]
    return (h_pos, h_neg)
